# baseline (speedup 1.0000x reference)
import functools

import jax
import jax.numpy as jnp
from jax import lax
from jax.experimental import pallas as pl
from jax.experimental.pallas import tpu as pltpu
from jax.experimental.pallas import tpu_sc as plsc

N_NODES = 10000
E_EDGES = 320000
DIM = 3
NC = 2
NS = 16
NW = NC * NS
EPW = E_EDGES // NW
CHUNK = 80
NCH = EPW // CHUNK
PAIRS = (NCH - 1) // 2
PROWS = CHUNK * 16 // 128
SUB_ROWS = N_NODES // NS
ROW_BLK = 1000
EROWS = E_EDGES * 16 // 128

_MESH = plsc.VectorSubcoreMesh(core_axis_name="c", subcore_axis_name="s")
_SC_PARAMS = pltpu.CompilerParams(use_tc_tiling_on_sc=False)


def _leaky(v):
    return jnp.where(v > 0, v, 0.2 * v)


def _dense_pre_body(hh_ref, pos_ref, wl_ref, bl_ref, wr_ref, br_ref, we_ref,
                    att_ref, mea_ref, a_ref, b_ref, xlp_ref):
    hh = hh_ref[...]
    xl = jnp.dot(hh, wl_ref[...], preferred_element_type=jnp.float32) + bl_ref[...]
    xr = jnp.dot(hh, wr_ref[...], preferred_element_type=jnp.float32) + br_ref[...]
    pw = jnp.dot(pos_ref[...], we_ref[...], preferred_element_type=jnp.float32)
    a_ref[...] = xl - pw
    b_ref[...] = xr + pw
    mew = jnp.dot(mea_ref[...], we_ref[...], preferred_element_type=jnp.float32)
    sv = xl + xr + mew
    slog = jnp.sum(_leaky(sv) * att_ref[...], axis=1, keepdims=True)
    rows = xl.shape[0]
    pad = jnp.zeros((rows, 14), jnp.float32)
    xlp = jnp.concatenate(
        [xl, jnp.ones((rows, 1), jnp.float32), slog, pad], axis=1)
    xlp_ref[...] = xlp


@functools.partial(jax.jit, static_argnames=("co",))
def _dense_pre(hh, pos, wl, bl, wr, br, we, att, mean_ea, co):
    ci = hh.shape[1]
    row_spec = lambda w: pl.BlockSpec((ROW_BLK, w), lambda i: (i, 0))
    full = lambda *shape: pl.BlockSpec(shape, lambda i: (0,) * len(shape))
    return pl.pallas_call(
        _dense_pre_body,
        grid=(N_NODES // ROW_BLK,),
        in_specs=[
            row_spec(ci), row_spec(DIM),
            full(ci, co), full(co), full(ci, co), full(co),
            full(DIM, co), full(1, co), full(1, DIM),
        ],
        out_specs=(row_spec(co), row_spec(co), row_spec(co + 16)),
        out_shape=(
            jax.ShapeDtypeStruct((N_NODES, co), jnp.float32),
            jax.ShapeDtypeStruct((N_NODES, co), jnp.float32),
            jax.ShapeDtypeStruct((N_NODES, co + 16), jnp.float32),
        ),
    )(hh, pos, wl, bl, wr, br, we, att.reshape(1, co), mean_ea)


def _mid_body(pacc_ref, slog_ref, exx_ref, gmax_ref):
    p = pacc_ref[...]
    r = lax.broadcasted_iota(jnp.int32, (128, 128), 0)
    c = lax.broadcasted_iota(jnp.int32, (128, 128), 1)
    pm = jnp.where((r // 16) == (c // 16), 1.0, 0.0).astype(jnp.float32)
    t = jnp.dot(p, pm, preferred_element_type=jnp.float32)
    g = jnp.maximum(jnp.max(t), jnp.max(slog_ref[...]))
    exx_ref[...] = jnp.exp(t - g)
    gmax_ref[...] = jnp.full((1, 1), g, jnp.float32)


@jax.jit
def _mid(pacc2d, slogp):
    return pl.pallas_call(
        _mid_body,
        out_shape=(
            jax.ShapeDtypeStruct((EROWS, 128), jnp.float32),
            jax.ShapeDtypeStruct((1, 1), jnp.float32),
        ),
    )(pacc2d, slogp)


def _final_body(acc_ref, xlp_ref, gmax_ref, bo_ref, h_ref, co):
    t = acc_ref[0] + acc_ref[1]
    xlp = xlp_ref[...]
    sex = jnp.exp(xlp[:, co + 1:co + 2] - gmax_ref[0, 0])
    t = t + sex * xlp
    den = t[:, co:co + 1]
    out = t[:, :co] / (den + 1e-16) + bo_ref[...]
    h_ref[...] = jnp.where(out > 0, out, jnp.exp(jnp.minimum(out, 0.0)) - 1.0)


@functools.partial(jax.jit, static_argnames=("co",))
def _final(acc, xlp, gmax, bo, co):
    cp = co + 16
    return pl.pallas_call(
        functools.partial(_final_body, co=co),
        grid=(N_NODES // ROW_BLK,),
        in_specs=[
            pl.BlockSpec((NC, ROW_BLK, cp), lambda i: (0, i, 0)),
            pl.BlockSpec((ROW_BLK, cp), lambda i: (i, 0)),
            pl.BlockSpec((1, 1), lambda i: (0, 0)),
            pl.BlockSpec((1, co), lambda i: (0, 0)),
        ],
        out_specs=pl.BlockSpec((ROW_BLK, co), lambda i: (i, 0)),
        out_shape=jax.ShapeDtypeStruct((N_NODES, co), jnp.float32),
    )(acc, xlp, gmax, bo.reshape(1, co))


def _make_mean_kernel():
    @functools.partial(
        pl.kernel, mesh=_MESH, compiler_params=_SC_PARAMS,
        out_type=jax.ShapeDtypeStruct((NW, 16), jnp.float32),
        scratch_types=[
            pltpu.VMEM((128,), jnp.int32),
            pltpu.VMEM((128,), jnp.int32),
            pltpu.VMEM((128, 16), jnp.float32),
            pltpu.VMEM((128, 16), jnp.float32),
            pltpu.VMEM((16,), jnp.float32),
            pltpu.SemaphoreType.DMA,
            pltpu.SemaphoreType.DMA,
        ],
    )
    def k(posp_hbm, src_hbm, dst_hbm, out_hbm,
          srcv, dstv, bufs, bufd, accv, sem1, sem2):
        wid = lax.axis_index("s") * NC + lax.axis_index("c")
        accv[...] = jnp.zeros((16,), jnp.float32)

        @pl.loop(wid, E_EDGES // 128, step=NW)
        def _chunk(kk):
            base = kk * 128
            pltpu.sync_copy(src_hbm.at[pl.ds(base, 128)], srcv)
            pltpu.sync_copy(dst_hbm.at[pl.ds(base, 128)], dstv)
            ca = pltpu.async_copy(posp_hbm.at[srcv], bufs, sem1)
            cb = pltpu.async_copy(posp_hbm.at[dstv], bufd, sem2)
            ca.wait()
            cb.wait()

            @pl.loop(0, 128)
            def _row(r):
                accv[...] = accv[...] + (bufd[r, :] - bufs[r, :])

        pltpu.sync_copy(accv, out_hbm.at[wid])

    return k


def _make_pass1(co):
    nj = co // 16

    @functools.partial(
        pl.kernel, mesh=_MESH, compiler_params=_SC_PARAMS,
        out_type=jax.ShapeDtypeStruct((EROWS, 128), jnp.float32),
        scratch_types=[
            pltpu.VMEM((EPW,), jnp.int32),
            pltpu.VMEM((EPW,), jnp.int32),
            [pltpu.VMEM((CHUNK, co), jnp.float32) for _ in range(2)],
            [pltpu.VMEM((CHUNK, co), jnp.float32) for _ in range(2)],
            [pltpu.VMEM((PROWS, 128), jnp.float32) for _ in range(2)],
            pltpu.VMEM((co,), jnp.float32),
            [pltpu.SemaphoreType.DMA for _ in range(2)],
            [pltpu.SemaphoreType.DMA for _ in range(2)],
            [pltpu.SemaphoreType.DMA for _ in range(2)],
        ],
    )
    def k(a_hbm, b_hbm, src_hbm, dst_hbm, att_hbm, out_hbm,
          srcall, dstall, bufa, bufb, paccv, attv, sema, semb, semw):
        cid = lax.axis_index("c")
        sid = lax.axis_index("s")
        wid = sid * NC + cid
        ebase = wid * EPW
        rbase = wid * (EPW * 16 // 128)
        pltpu.sync_copy(att_hbm, attv)
        pltpu.sync_copy(src_hbm.at[pl.ds(ebase, EPW)], srcall)
        pltpu.sync_copy(dst_hbm.at[pl.ds(ebase, EPW)], dstall)
        att6 = [attv[pl.ds(j * 16, 16)] * 0.6 for j in range(nj)]
        att4 = [attv[pl.ds(j * 16, 16)] * 0.4 for j in range(nj)]

        def ga(i, s):
            return pltpu.make_async_copy(
                a_hbm.at[srcall.at[pl.ds(i * CHUNK, CHUNK)]], bufa[s], sema[s])

        def gb(i, s):
            return pltpu.make_async_copy(
                b_hbm.at[dstall.at[pl.ds(i * CHUNK, CHUNK)]], bufb[s], semb[s])

        def wo(i, s):
            return pltpu.make_async_copy(
                paccv[s], out_hbm.at[pl.ds(rbase + i * PROWS, PROWS)], semw[s])

        def issue(i, s):
            ga(i, s).start()
            gb(i, s).start()

        def compute(i, s):
            ga(i, s).wait()
            gb(i, s).wait()

            @pl.loop(0, CHUNK)
            def _edge(e):
                acc = jnp.zeros((16,), jnp.float32)
                for j in range(nj):
                    v = bufa[s][e, pl.ds(j * 16, 16)] + bufb[s][e, pl.ds(j * 16, 16)]
                    acc = acc + v * att6[j] + jnp.abs(v) * att4[j]
                paccv[s][e // 8, pl.ds((e % 8) * 16, 16)] = acc

            wo(i, s).start()

        issue(0, 0)
        issue(1, 1)
        compute(0, 0)
        issue(2, 0)
        compute(1, 1)

        @pl.loop(1, PAIRS)
        def _pair(p):
            i = 2 * p
            issue(i + 1, 1)
            wo(i - 2, 0).wait()
            compute(i, 0)
            issue(i + 2, 0)
            wo(i - 1, 1).wait()
            compute(i + 1, 1)

        wo(NCH - 3, 0).wait()
        compute(NCH - 1, 0)
        wo(NCH - 2, 1).wait()
        wo(NCH - 1, 0).wait()

    return k


def _make_pass2(co):
    cp = co + 16
    nj = cp // 16
    ch2 = 40
    pr2 = ch2 * 16 // 128
    nch2 = EPW // ch2

    @functools.partial(
        pl.kernel, mesh=_MESH, compiler_params=_SC_PARAMS,
        out_type=jax.ShapeDtypeStruct((NC, N_NODES, cp), jnp.float32),
        scratch_types=[
            pltpu.VMEM((EPW,), jnp.int32),
            [pltpu.VMEM((ch2,), jnp.int32) for _ in range(2)],
            [pltpu.VMEM((ch2,), jnp.int32) for _ in range(2)],
            [pltpu.VMEM((pr2, 128), jnp.float32) for _ in range(2)],
            [pltpu.VMEM((ch2, cp), jnp.float32) for _ in range(2)],
            [pltpu.VMEM((ch2, cp), jnp.float32) for _ in range(2)],
            pltpu.VMEM_SHARED((N_NODES, cp), jnp.float32),
            [pltpu.SemaphoreType.DMA for _ in range(2)],
            [pltpu.SemaphoreType.DMA for _ in range(2)],
            [pltpu.SemaphoreType.DMA for _ in range(2)],
            [pltpu.SemaphoreType.DMA for _ in range(2)],
        ],
    )
    def k(xlp_hbm, src_hbm, dst_hbm, exx_hbm, zeros_hbm, out_hbm,
          srcall, dstv, sdst, exv, rows, srows, acc_sh, semg, seme, semd, sems):
        cid = lax.axis_index("c")
        sid = lax.axis_index("s")
        wid = sid * NC + cid
        ebase = wid * EPW
        rbase = wid * (EPW * 16 // 128)
        r0 = sid * SUB_ROWS
        pltpu.sync_copy(src_hbm.at[pl.ds(ebase, EPW)], srcall)
        pltpu.sync_copy(zeros_hbm.at[pl.ds(r0, SUB_ROWS)],
                        acc_sh.at[pl.ds(r0, SUB_ROWS)])
        plsc.subcore_barrier()

        def gr(i, s):
            return pltpu.make_async_copy(
                xlp_hbm.at[srcall.at[pl.ds(i * ch2, ch2)]], rows[s], semg[s])

        def ge(i, s):
            return pltpu.make_async_copy(
                exx_hbm.at[pl.ds(rbase + i * pr2, pr2)], exv[s], seme[s])

        def gd(i, s):
            return pltpu.make_async_copy(
                dst_hbm.at[pl.ds(ebase + i * ch2, ch2)], dstv[s], semd[s])

        def sc_start(s):
            pltpu.async_copy(srows[s], acc_sh.at[sdst[s]], sems[s], add=True)

        def sc_wait(s):
            pltpu.make_async_copy(srows[s], acc_sh.at[sdst[s]], sems[s]).wait()

        def issue(i, s):
            gr(i, s).start()
            ge(i, s).start()
            gd(i, s).start()

        def compute(i, s, first):
            gr(i, s).wait()
            ge(i, s).wait()
            gd(i, s).wait()
            if not first:
                sc_wait(s)
            sdst[s][pl.ds(0, 16)] = dstv[s][pl.ds(0, 16)]
            sdst[s][pl.ds(16, 16)] = dstv[s][pl.ds(16, 16)]
            sdst[s][pl.ds(24, 16)] = dstv[s][pl.ds(24, 16)]

            @pl.loop(0, ch2)
            def _edge(e):
                exs = exv[s][e // 8, pl.ds((e % 8) * 16, 16)]
                for j in range(nj):
                    srows[s][e, pl.ds(j * 16, 16)] = (
                        rows[s][e, pl.ds(j * 16, 16)] * exs)

            sc_start(s)

        issue(0, 0)
        issue(1, 1)
        compute(0, 0, True)
        issue(2, 0)
        compute(1, 1, True)

        @pl.loop(1, (nch2 - 2) // 2)
        def _pair(p):
            i = 2 * p
            issue(i + 1, 1)
            compute(i, 0, False)
            issue(i + 2, 0)
            compute(i + 1, 1, False)

        issue(nch2 - 1, 1)
        compute(nch2 - 2, 0, False)
        compute(nch2 - 1, 1, False)
        sc_wait(0)
        sc_wait(1)
        plsc.subcore_barrier()
        pltpu.sync_copy(acc_sh.at[pl.ds(r0, SUB_ROWS)],
                        out_hbm.at[cid, pl.ds(r0, SUB_ROWS)])

    return k


_MEAN_K = _make_mean_kernel()
_PASS1 = {128: _make_pass1(128), 64: _make_pass1(64)}
_PASS2 = {128: _make_pass2(128), 64: _make_pass2(64)}


def _layer(hh, pos, src, dst, mean_ea, zeros_cp, wl, bl, wr, br, we, att, bo):
    co = wl.shape[1]
    a, b, xlp = _dense_pre(hh, pos, wl, bl, wr, br, we, att, mean_ea, co)
    pacc = _PASS1[co](a, b, src, dst, att)
    slog = xlp[:, co + 1]
    slogp = jnp.pad(slog, (0, 240), constant_values=-1e30).reshape(80, 128)
    exx, gmax = _mid(pacc, slogp)
    acc = _PASS2[co](xlp, src, dst, exx, zeros_cp)
    return _final(acc, xlp, gmax, bo, co)


def kernel(x, edge_index, pos,
           Wl0, bl0, Wr0, br0, We0, att0, bo0,
           Wl1, bl1, Wr1, br1, We1, att1, bo1,
           Wl2, bl2, Wr2, br2, We2, att2, bo2):
    src = edge_index[0]
    dst = edge_index[1]
    posp = jnp.pad(pos, ((0, 0), (0, 13)))
    sums = _MEAN_K(posp, src, dst)
    mean_ea = (jnp.sum(sums, axis=0)[:DIM] / E_EDGES).reshape(1, DIM)
    z144 = jnp.zeros((N_NODES, 144), jnp.float32)
    z80 = jnp.zeros((N_NODES, 80), jnp.float32)
    h = jnp.concatenate([x, pos], axis=1)
    h = _layer(h, pos, src, dst, mean_ea, z144,
               Wl0, bl0, Wr0, br0, We0, att0, bo0)
    h = jnp.concatenate([h, pos], axis=1)
    h = _layer(h, pos, src, dst, mean_ea, z144,
               Wl1, bl1, Wr1, br1, We1, att1, bo1)
    h = jnp.concatenate([h, pos], axis=1)
    h = _layer(h, pos, src, dst, mean_ea, z80,
               Wl2, bl2, Wr2, br2, We2, att2, bo2)
    return (h, edge_index, pos)


# speedup vs baseline: 1.4008x; 1.4008x over previous
import functools

import jax
import jax.numpy as jnp
from jax import lax
from jax.experimental import pallas as pl
from jax.experimental.pallas import tpu as pltpu
from jax.experimental.pallas import tpu_sc as plsc

N_NODES = 10000
E_EDGES = 320000
DIM = 3
NC = 2
NS = 16
NW = NC * NS
EPW = E_EDGES // NW
CHUNK = 80
NCH = EPW // CHUNK
PAIRS = (NCH - 1) // 2
PROWS = CHUNK * 16 // 128
SUB_ROWS = N_NODES // NS
ROW_BLK = 1000
EROWS = E_EDGES * 16 // 128

_MESH = plsc.VectorSubcoreMesh(core_axis_name="c", subcore_axis_name="s")
_SC_PARAMS = pltpu.CompilerParams(use_tc_tiling_on_sc=False)


def _leaky(v):
    return jnp.where(v > 0, v, 0.2 * v)


def _dense_pre_body(hh_ref, pos_ref, wl_ref, bl_ref, wr_ref, br_ref, we_ref,
                    att_ref, mea_ref, a_ref, b_ref, xlp_ref):
    hh = hh_ref[...]
    xl = jnp.dot(hh, wl_ref[...], preferred_element_type=jnp.float32) + bl_ref[...]
    xr = jnp.dot(hh, wr_ref[...], preferred_element_type=jnp.float32) + br_ref[...]
    pw = jnp.dot(pos_ref[...], we_ref[...], preferred_element_type=jnp.float32)
    a_ref[...] = xl - pw
    b_ref[...] = xr + pw
    mew = jnp.dot(mea_ref[...], we_ref[...], preferred_element_type=jnp.float32)
    sv = xl + xr + mew
    slog = jnp.sum(_leaky(sv) * att_ref[...], axis=1, keepdims=True)
    rows = xl.shape[0]
    pad = jnp.zeros((rows, 14), jnp.float32)
    xlp = jnp.concatenate(
        [xl, jnp.ones((rows, 1), jnp.float32), slog, pad], axis=1)
    xlp_ref[...] = xlp


@functools.partial(jax.jit, static_argnames=("co",))
def _dense_pre(hh, pos, wl, bl, wr, br, we, att, mean_ea, co):
    ci = hh.shape[1]
    row_spec = lambda w: pl.BlockSpec((ROW_BLK, w), lambda i: (i, 0))
    full = lambda *shape: pl.BlockSpec(shape, lambda i: (0,) * len(shape))
    return pl.pallas_call(
        _dense_pre_body,
        grid=(N_NODES // ROW_BLK,),
        in_specs=[
            row_spec(ci), row_spec(DIM),
            full(ci, co), full(co), full(ci, co), full(co),
            full(DIM, co), full(1, co), full(1, DIM),
        ],
        out_specs=(row_spec(co), row_spec(co), row_spec(co + 16)),
        out_shape=(
            jax.ShapeDtypeStruct((N_NODES, co), jnp.float32),
            jax.ShapeDtypeStruct((N_NODES, co), jnp.float32),
            jax.ShapeDtypeStruct((N_NODES, co + 16), jnp.float32),
        ),
    )(hh, pos, wl, bl, wr, br, we, att.reshape(1, co), mean_ea)


def _mid_body(pacc_ref, slog_ref, exx_ref, gmax_ref):
    p = pacc_ref[...]
    r = lax.broadcasted_iota(jnp.int32, (128, 128), 0)
    c = lax.broadcasted_iota(jnp.int32, (128, 128), 1)
    pm = jnp.where((r // 16) == (c // 16), 1.0, 0.0).astype(jnp.float32)
    t = jnp.dot(p, pm, preferred_element_type=jnp.float32)
    g = jnp.maximum(jnp.max(t), jnp.max(slog_ref[...]))
    exx_ref[...] = jnp.exp(t - g)
    gmax_ref[...] = jnp.full((1, 1), g, jnp.float32)


@jax.jit
def _mid(pacc2d, slogp):
    return pl.pallas_call(
        _mid_body,
        out_shape=(
            jax.ShapeDtypeStruct((EROWS, 128), jnp.float32),
            jax.ShapeDtypeStruct((1, 1), jnp.float32),
        ),
    )(pacc2d, slogp)


def _final_body(acc_ref, xlp_ref, gmax_ref, bo_ref, h_ref, co):
    t = acc_ref[0] + acc_ref[1]
    xlp = xlp_ref[...]
    sex = jnp.exp(xlp[:, co + 1:co + 2] - gmax_ref[0, 0])
    t = t + sex * xlp
    den = t[:, co:co + 1]
    out = t[:, :co] / (den + 1e-16) + bo_ref[...]
    h_ref[...] = jnp.where(out > 0, out, jnp.exp(jnp.minimum(out, 0.0)) - 1.0)


@functools.partial(jax.jit, static_argnames=("co",))
def _final(acc, xlp, gmax, bo, co):
    cp = co + 16
    return pl.pallas_call(
        functools.partial(_final_body, co=co),
        grid=(N_NODES // ROW_BLK,),
        in_specs=[
            pl.BlockSpec((NC, ROW_BLK, cp), lambda i: (0, i, 0)),
            pl.BlockSpec((ROW_BLK, cp), lambda i: (i, 0)),
            pl.BlockSpec((1, 1), lambda i: (0, 0)),
            pl.BlockSpec((1, co), lambda i: (0, 0)),
        ],
        out_specs=pl.BlockSpec((ROW_BLK, co), lambda i: (i, 0)),
        out_shape=jax.ShapeDtypeStruct((N_NODES, co), jnp.float32),
    )(acc, xlp, gmax, bo.reshape(1, co))


def _make_mean_kernel():
    @functools.partial(
        pl.kernel, mesh=_MESH, compiler_params=_SC_PARAMS,
        out_type=jax.ShapeDtypeStruct((NW, 16), jnp.float32),
        scratch_types=[
            pltpu.VMEM((128,), jnp.int32),
            pltpu.VMEM((128,), jnp.int32),
            pltpu.VMEM((128, 16), jnp.float32),
            pltpu.VMEM((128, 16), jnp.float32),
            pltpu.VMEM((16,), jnp.float32),
            pltpu.SemaphoreType.DMA,
            pltpu.SemaphoreType.DMA,
        ],
    )
    def k(posp_hbm, src_hbm, dst_hbm, out_hbm,
          srcv, dstv, bufs, bufd, accv, sem1, sem2):
        wid = lax.axis_index("s") * NC + lax.axis_index("c")
        accv[...] = jnp.zeros((16,), jnp.float32)

        @pl.loop(wid, E_EDGES // 128, step=NW)
        def _chunk(kk):
            base = kk * 128
            pltpu.sync_copy(src_hbm.at[pl.ds(base, 128)], srcv)
            pltpu.sync_copy(dst_hbm.at[pl.ds(base, 128)], dstv)
            ca = pltpu.async_copy(posp_hbm.at[srcv], bufs, sem1)
            cb = pltpu.async_copy(posp_hbm.at[dstv], bufd, sem2)
            ca.wait()
            cb.wait()

            @pl.loop(0, 128)
            def _row(r):
                accv[...] = accv[...] + (bufd[r, :] - bufs[r, :])

        pltpu.sync_copy(accv, out_hbm.at[wid])

    return k


def _make_pass1(co):
    nj = co // 16
    ch1 = 128
    nf = EPW // ch1
    tl = EPW - nf * ch1
    pr1 = ch1 * 16 // 128
    trow = tl * 16 // 128

    @functools.partial(
        pl.kernel, mesh=_MESH, compiler_params=_SC_PARAMS,
        out_type=jax.ShapeDtypeStruct((EROWS, 128), jnp.float32),
        scratch_types=[
            pltpu.VMEM((EPW,), jnp.int32),
            pltpu.VMEM((EPW,), jnp.int32),
            [pltpu.VMEM((ch1, co), jnp.float32) for _ in range(2)],
            [pltpu.VMEM((ch1, co), jnp.float32) for _ in range(2)],
            [pltpu.VMEM((pr1, 128), jnp.float32) for _ in range(2)],
            pltpu.VMEM((co,), jnp.float32),
            [pltpu.SemaphoreType.DMA for _ in range(2)],
            [pltpu.SemaphoreType.DMA for _ in range(2)],
            [pltpu.SemaphoreType.DMA for _ in range(2)],
        ],
    )
    def k(a_hbm, b_hbm, src_hbm, dst_hbm, att_hbm, out_hbm,
          srcall, dstall, bufa, bufb, paccv, attv, sema, semb, semw):
        cid = lax.axis_index("c")
        sid = lax.axis_index("s")
        wid = sid * NC + cid
        ebase = wid * EPW
        rbase = wid * (EPW * 16 // 128)
        pltpu.sync_copy(att_hbm, attv)
        pltpu.sync_copy(src_hbm.at[pl.ds(ebase, EPW)], srcall)
        pltpu.sync_copy(dst_hbm.at[pl.ds(ebase, EPW)], dstall)
        att6 = [attv[pl.ds(j * 16, 16)] * 0.6 for j in range(nj)]
        att4 = [attv[pl.ds(j * 16, 16)] * 0.4 for j in range(nj)]

        def ga(i, s):
            return pltpu.make_async_copy(
                a_hbm.at[srcall.at[pl.ds(i * ch1, ch1)]], bufa[s], sema[s])

        def gb(i, s):
            return pltpu.make_async_copy(
                b_hbm.at[dstall.at[pl.ds(i * ch1, ch1)]], bufb[s], semb[s])

        def wo(i, s):
            return pltpu.make_async_copy(
                paccv[s], out_hbm.at[pl.ds(rbase + i * pr1, pr1)], semw[s])

        def issue(i, s):
            ga(i, s).start()
            gb(i, s).start()

        def edge_loop(n, s):
            @pl.loop(0, n)
            def _edge(e):
                acc = jnp.zeros((16,), jnp.float32)
                for j in range(nj):
                    v = bufa[s][e, pl.ds(j * 16, 16)] + bufb[s][e, pl.ds(j * 16, 16)]
                    acc = acc + v * att6[j] + jnp.abs(v) * att4[j]
                paccv[s][e // 8, pl.ds((e % 8) * 16, 16)] = acc

        def compute(i, s):
            ga(i, s).wait()
            gb(i, s).wait()
            edge_loop(ch1, s)
            wo(i, s).start()

        issue(0, 0)
        issue(1, 1)
        compute(0, 0)
        issue(2, 0)
        compute(1, 1)

        @pl.loop(1, nf // 2)
        def _pair(p):
            i = 2 * p
            issue(i + 1, 1)
            wo(i - 2, 0).wait()
            compute(i, 0)
            issue(jnp.minimum(i + 2, nf - 1), 0)
            wo(i - 1, 1).wait()
            compute(i + 1, 1)

        ga(nf - 1, 0).wait()
        gb(nf - 1, 0).wait()
        pltpu.make_async_copy(
            a_hbm.at[srcall.at[pl.ds(nf * ch1, tl)]],
            bufa[0].at[pl.ds(0, tl)], sema[0]).start()
        pltpu.make_async_copy(
            b_hbm.at[dstall.at[pl.ds(nf * ch1, tl)]],
            bufb[0].at[pl.ds(0, tl)], semb[0]).start()
        wo(nf - 2, 0).wait()
        pltpu.make_async_copy(
            a_hbm.at[srcall.at[pl.ds(nf * ch1, tl)]],
            bufa[0].at[pl.ds(0, tl)], sema[0]).wait()
        pltpu.make_async_copy(
            b_hbm.at[dstall.at[pl.ds(nf * ch1, tl)]],
            bufb[0].at[pl.ds(0, tl)], semb[0]).wait()
        edge_loop(tl, 0)
        pltpu.make_async_copy(
            paccv[0].at[pl.ds(0, trow)],
            out_hbm.at[pl.ds(rbase + nf * pr1, trow)], semw[0]).start()
        wo(nf - 1, 1).wait()
        pltpu.make_async_copy(
            paccv[0].at[pl.ds(0, trow)],
            out_hbm.at[pl.ds(rbase + nf * pr1, trow)], semw[0]).wait()

    return k


def _make_pass2(co):
    cp = co + 16
    nj = cp // 16

    @functools.partial(
        pl.kernel, mesh=_MESH, compiler_params=_SC_PARAMS,
        out_type=jax.ShapeDtypeStruct((NC, N_NODES, cp), jnp.float32),
        scratch_types=[
            pltpu.VMEM((EPW,), jnp.int32),
            [pltpu.VMEM((CHUNK,), jnp.int32) for _ in range(2)],
            [pltpu.VMEM((PROWS, 128), jnp.float32) for _ in range(2)],
            [pltpu.VMEM((CHUNK, cp), jnp.float32) for _ in range(2)],
            pltpu.VMEM_SHARED((N_NODES, cp), jnp.float32),
            [pltpu.SemaphoreType.DMA for _ in range(2)],
            [pltpu.SemaphoreType.DMA for _ in range(2)],
            [pltpu.SemaphoreType.DMA for _ in range(2)],
        ],
    )
    def k(xlp_hbm, src_hbm, dst_hbm, exx_hbm, zeros_hbm, out_hbm,
          srcall, dstv, exv, rows, acc_sh, semg, seme, semd):
        cid = lax.axis_index("c")
        sid = lax.axis_index("s")
        wid = sid * NC + cid
        ebase = wid * EPW
        rbase = wid * (EPW * 16 // 128)
        r0 = sid * SUB_ROWS
        pltpu.sync_copy(src_hbm.at[pl.ds(ebase, EPW)], srcall)
        pltpu.sync_copy(zeros_hbm.at[pl.ds(r0, SUB_ROWS)],
                        acc_sh.at[pl.ds(r0, SUB_ROWS)])
        plsc.subcore_barrier()

        def gr(i, s):
            return pltpu.make_async_copy(
                xlp_hbm.at[srcall.at[pl.ds(i * CHUNK, CHUNK)]], rows[s], semg[s])

        def ge(i, s):
            return pltpu.make_async_copy(
                exx_hbm.at[pl.ds(rbase + i * PROWS, PROWS)], exv[s], seme[s])

        def gd(i, s):
            return pltpu.make_async_copy(
                dst_hbm.at[pl.ds(ebase + i * CHUNK, CHUNK)], dstv[s], semd[s])

        def issue(i, s):
            gr(i, s).start()
            ge(i, s).start()
            gd(i, s).start()

        def compute(i, s):
            gr(i, s).wait()
            ge(i, s).wait()
            gd(i, s).wait()

            @pl.loop(0, CHUNK)
            def _edge(e):
                exs = exv[s][e // 8, pl.ds((e % 8) * 16, 16)]
                for j in range(nj):
                    rows[s][e, pl.ds(j * 16, 16)] = rows[s][e, pl.ds(j * 16, 16)] * exs

            pltpu.sync_copy(rows[s], acc_sh.at[dstv[s]], add=True)

        issue(0, 0)
        issue(1, 1)
        compute(0, 0)
        issue(2, 0)
        compute(1, 1)

        @pl.loop(1, PAIRS)
        def _pair(p):
            i = 2 * p
            issue(i + 1, 1)
            compute(i, 0)
            issue(i + 2, 0)
            compute(i + 1, 1)

        compute(NCH - 1, 0)
        plsc.subcore_barrier()
        pltpu.sync_copy(acc_sh.at[pl.ds(r0, SUB_ROWS)],
                        out_hbm.at[cid, pl.ds(r0, SUB_ROWS)])

    return k


_MEAN_K = _make_mean_kernel()
_PASS1 = {128: _make_pass1(128), 64: _make_pass1(64)}
_PASS2 = {128: _make_pass2(128), 64: _make_pass2(64)}


def _layer(hh, pos, src, dst, mean_ea, zeros_cp, wl, bl, wr, br, we, att, bo):
    co = wl.shape[1]
    a, b, xlp = _dense_pre(hh, pos, wl, bl, wr, br, we, att, mean_ea, co)
    pacc = _PASS1[co](a, b, src, dst, att)
    slog = xlp[:, co + 1]
    slogp = jnp.pad(slog, (0, 240), constant_values=-1e30).reshape(80, 128)
    exx, gmax = _mid(pacc, slogp)
    acc = _PASS2[co](xlp, src, dst, exx, zeros_cp)
    return _final(acc, xlp, gmax, bo, co)


def kernel(x, edge_index, pos,
           Wl0, bl0, Wr0, br0, We0, att0, bo0,
           Wl1, bl1, Wr1, br1, We1, att1, bo1,
           Wl2, bl2, Wr2, br2, We2, att2, bo2):
    src = edge_index[0]
    dst = edge_index[1]
    posp = jnp.pad(pos, ((0, 0), (0, 13)))
    sums = _MEAN_K(posp, src, dst)
    mean_ea = (jnp.sum(sums, axis=0)[:DIM] / E_EDGES).reshape(1, DIM)
    z144 = jnp.zeros((N_NODES, 144), jnp.float32)
    z80 = jnp.zeros((N_NODES, 80), jnp.float32)
    h = jnp.concatenate([x, pos], axis=1)
    h = _layer(h, pos, src, dst, mean_ea, z144,
               Wl0, bl0, Wr0, br0, We0, att0, bo0)
    h = jnp.concatenate([h, pos], axis=1)
    h = _layer(h, pos, src, dst, mean_ea, z144,
               Wl1, bl1, Wr1, br1, We1, att1, bo1)
    h = jnp.concatenate([h, pos], axis=1)
    h = _layer(h, pos, src, dst, mean_ea, z80,
               Wl2, bl2, Wr2, br2, We2, att2, bo2)
    return (h, edge_index, pos)


# speedup vs baseline: 1.4278x; 1.0193x over previous
import functools

import jax
import jax.numpy as jnp
from jax import lax
from jax.experimental import pallas as pl
from jax.experimental.pallas import tpu as pltpu
from jax.experimental.pallas import tpu_sc as plsc

N_NODES = 10000
E_EDGES = 320000
DIM = 3
NC = 2
NS = 16
NW = NC * NS
EPW = E_EDGES // NW
CHUNK = 80
NCH = EPW // CHUNK
PAIRS = (NCH - 1) // 2
PROWS = CHUNK * 16 // 128
SUB_ROWS = N_NODES // NS
ROW_BLK = 1000
EROWS = E_EDGES * 16 // 128

_MESH = plsc.VectorSubcoreMesh(core_axis_name="c", subcore_axis_name="s")
_SC_PARAMS = pltpu.CompilerParams(use_tc_tiling_on_sc=False)


def _leaky(v):
    return jnp.where(v > 0, v, 0.2 * v)


def _dense_pre_body(hh_ref, pos_ref, wl_ref, bl_ref, wr_ref, br_ref, we_ref,
                    att_ref, mea_ref, a_ref, b_ref, xlp_ref):
    hh = hh_ref[...]
    xl = jnp.dot(hh, wl_ref[...], preferred_element_type=jnp.float32) + bl_ref[...]
    xr = jnp.dot(hh, wr_ref[...], preferred_element_type=jnp.float32) + br_ref[...]
    pw = jnp.dot(pos_ref[...], we_ref[...], preferred_element_type=jnp.float32)
    a_ref[...] = xl - pw
    b_ref[...] = xr + pw
    mew = jnp.dot(mea_ref[...], we_ref[...], preferred_element_type=jnp.float32)
    sv = xl + xr + mew
    slog = jnp.sum(_leaky(sv) * att_ref[...], axis=1, keepdims=True)
    rows = xl.shape[0]
    pad = jnp.zeros((rows, 14), jnp.float32)
    xlp = jnp.concatenate(
        [xl, jnp.ones((rows, 1), jnp.float32), slog, pad], axis=1)
    xlp_ref[...] = xlp


@functools.partial(jax.jit, static_argnames=("co",))
def _dense_pre(hh, pos, wl, bl, wr, br, we, att, mean_ea, co):
    ci = hh.shape[1]
    row_spec = lambda w: pl.BlockSpec((ROW_BLK, w), lambda i: (i, 0))
    full = lambda *shape: pl.BlockSpec(shape, lambda i: (0,) * len(shape))
    return pl.pallas_call(
        _dense_pre_body,
        grid=(N_NODES // ROW_BLK,),
        in_specs=[
            row_spec(ci), row_spec(DIM),
            full(ci, co), full(co), full(ci, co), full(co),
            full(DIM, co), full(1, co), full(1, DIM),
        ],
        out_specs=(row_spec(co), row_spec(co), row_spec(co + 16)),
        out_shape=(
            jax.ShapeDtypeStruct((N_NODES, co), jnp.float32),
            jax.ShapeDtypeStruct((N_NODES, co), jnp.float32),
            jax.ShapeDtypeStruct((N_NODES, co + 16), jnp.float32),
        ),
    )(hh, pos, wl, bl, wr, br, we, att.reshape(1, co), mean_ea)


def _mid_body(pacc_ref, slog_ref, exx_ref, gmax_ref):
    p = pacc_ref[...]
    r = lax.broadcasted_iota(jnp.int32, (128, 128), 0)
    c = lax.broadcasted_iota(jnp.int32, (128, 128), 1)
    pm = jnp.where((r // 16) == (c // 16), 1.0, 0.0).astype(jnp.float32)
    t = jnp.dot(p, pm, preferred_element_type=jnp.float32)
    g = jnp.maximum(jnp.max(t), jnp.max(slog_ref[...]))
    exx_ref[...] = jnp.exp(t - g)
    gmax_ref[...] = jnp.full((1, 1), g, jnp.float32)


@jax.jit
def _mid(pacc2d, slogp):
    return pl.pallas_call(
        _mid_body,
        out_shape=(
            jax.ShapeDtypeStruct((EROWS, 128), jnp.float32),
            jax.ShapeDtypeStruct((1, 1), jnp.float32),
        ),
    )(pacc2d, slogp)


def _final_body(acc_ref, xlp_ref, gmax_ref, bo_ref, h_ref, co):
    t = acc_ref[0] + acc_ref[1]
    xlp = xlp_ref[...]
    sex = jnp.exp(xlp[:, co + 1:co + 2] - gmax_ref[0, 0])
    t = t + sex * xlp
    den = t[:, co:co + 1]
    out = t[:, :co] / (den + 1e-16) + bo_ref[...]
    h_ref[...] = jnp.where(out > 0, out, jnp.exp(jnp.minimum(out, 0.0)) - 1.0)


@functools.partial(jax.jit, static_argnames=("co",))
def _final(acc, xlp, gmax, bo, co):
    cp = co + 16
    return pl.pallas_call(
        functools.partial(_final_body, co=co),
        grid=(N_NODES // ROW_BLK,),
        in_specs=[
            pl.BlockSpec((NC, ROW_BLK, cp), lambda i: (0, i, 0)),
            pl.BlockSpec((ROW_BLK, cp), lambda i: (i, 0)),
            pl.BlockSpec((1, 1), lambda i: (0, 0)),
            pl.BlockSpec((1, co), lambda i: (0, 0)),
        ],
        out_specs=pl.BlockSpec((ROW_BLK, co), lambda i: (i, 0)),
        out_shape=jax.ShapeDtypeStruct((N_NODES, co), jnp.float32),
    )(acc, xlp, gmax, bo.reshape(1, co))


def _fused_body(acc_ref, xlp_ref, gmax_ref, bo_ref, pos_ref,
                wl_ref, bl_ref, wr_ref, br_ref, we_ref, att_ref, mea_ref,
                a_ref, b_ref, xlp2_ref, co):
    t = acc_ref[0] + acc_ref[1]
    xlp = xlp_ref[...]
    sex = jnp.exp(xlp[:, co + 1:co + 2] - gmax_ref[0, 0])
    t = t + sex * xlp
    den = t[:, co:co + 1]
    out = t[:, :co] / (den + 1e-16) + bo_ref[...]
    h = jnp.where(out > 0, out, jnp.exp(jnp.minimum(out, 0.0)) - 1.0)
    hh = jnp.concatenate([h, pos_ref[...]], axis=1)
    xl = jnp.dot(hh, wl_ref[...], preferred_element_type=jnp.float32) + bl_ref[...]
    xr = jnp.dot(hh, wr_ref[...], preferred_element_type=jnp.float32) + br_ref[...]
    pw = jnp.dot(pos_ref[...], we_ref[...], preferred_element_type=jnp.float32)
    a_ref[...] = xl - pw
    b_ref[...] = xr + pw
    mew = jnp.dot(mea_ref[...], we_ref[...], preferred_element_type=jnp.float32)
    sv = xl + xr + mew
    slog = jnp.sum(_leaky(sv) * att_ref[...], axis=1, keepdims=True)
    rows = xl.shape[0]
    pad = jnp.zeros((rows, 14), jnp.float32)
    xlp2_ref[...] = jnp.concatenate(
        [xl, jnp.ones((rows, 1), jnp.float32), slog, pad], axis=1)


@functools.partial(jax.jit, static_argnames=("co", "co2"))
def _fused(acc, xlp, gmax, bo, pos, wl, bl, wr, br, we, att, mean_ea, co, co2):
    cp = co + 16
    ci2 = co + DIM
    row_spec = lambda w: pl.BlockSpec((ROW_BLK, w), lambda i: (i, 0))
    full = lambda *shape: pl.BlockSpec(shape, lambda i: (0,) * len(shape))
    return pl.pallas_call(
        functools.partial(_fused_body, co=co),
        grid=(N_NODES // ROW_BLK,),
        in_specs=[
            pl.BlockSpec((NC, ROW_BLK, cp), lambda i: (0, i, 0)),
            row_spec(cp),
            full(1, 1),
            full(1, co),
            row_spec(DIM),
            full(ci2, co2), full(co2), full(ci2, co2), full(co2),
            full(DIM, co2), full(1, co2), full(1, DIM),
        ],
        out_specs=(row_spec(co2), row_spec(co2), row_spec(co2 + 16)),
        out_shape=(
            jax.ShapeDtypeStruct((N_NODES, co2), jnp.float32),
            jax.ShapeDtypeStruct((N_NODES, co2), jnp.float32),
            jax.ShapeDtypeStruct((N_NODES, co2 + 16), jnp.float32),
        ),
    )(acc, xlp, gmax, bo.reshape(1, co), pos,
      wl, bl, wr, br, we, att.reshape(1, co2), mean_ea)


def _make_mean_kernel():
    @functools.partial(
        pl.kernel, mesh=_MESH, compiler_params=_SC_PARAMS,
        out_type=jax.ShapeDtypeStruct((NW, 16), jnp.float32),
        scratch_types=[
            pltpu.VMEM((128,), jnp.int32),
            pltpu.VMEM((128,), jnp.int32),
            pltpu.VMEM((128, 16), jnp.float32),
            pltpu.VMEM((128, 16), jnp.float32),
            pltpu.VMEM((16,), jnp.float32),
            pltpu.SemaphoreType.DMA,
            pltpu.SemaphoreType.DMA,
        ],
    )
    def k(posp_hbm, src_hbm, dst_hbm, out_hbm,
          srcv, dstv, bufs, bufd, accv, sem1, sem2):
        wid = lax.axis_index("s") * NC + lax.axis_index("c")
        accv[...] = jnp.zeros((16,), jnp.float32)

        @pl.loop(wid, E_EDGES // 128, step=NW)
        def _chunk(kk):
            base = kk * 128
            pltpu.sync_copy(src_hbm.at[pl.ds(base, 128)], srcv)
            pltpu.sync_copy(dst_hbm.at[pl.ds(base, 128)], dstv)
            ca = pltpu.async_copy(posp_hbm.at[srcv], bufs, sem1)
            cb = pltpu.async_copy(posp_hbm.at[dstv], bufd, sem2)
            ca.wait()
            cb.wait()

            @pl.loop(0, 128)
            def _row(r):
                accv[...] = accv[...] + (bufd[r, :] - bufs[r, :])

        pltpu.sync_copy(accv, out_hbm.at[wid])

    return k


def _make_pass1(co):
    nj = co // 16
    ch1 = 128
    nf = EPW // ch1
    tl = EPW - nf * ch1
    pr1 = ch1 * 16 // 128
    trow = tl * 16 // 128

    @functools.partial(
        pl.kernel, mesh=_MESH, compiler_params=_SC_PARAMS,
        out_type=jax.ShapeDtypeStruct((EROWS, 128), jnp.float32),
        scratch_types=[
            pltpu.VMEM((EPW,), jnp.int32),
            pltpu.VMEM((EPW,), jnp.int32),
            [pltpu.VMEM((ch1, co), jnp.float32) for _ in range(2)],
            [pltpu.VMEM((ch1, co), jnp.float32) for _ in range(2)],
            [pltpu.VMEM((pr1, 128), jnp.float32) for _ in range(2)],
            pltpu.VMEM((co,), jnp.float32),
            [pltpu.SemaphoreType.DMA for _ in range(2)],
            [pltpu.SemaphoreType.DMA for _ in range(2)],
            [pltpu.SemaphoreType.DMA for _ in range(2)],
        ],
    )
    def k(a_hbm, b_hbm, src_hbm, dst_hbm, att_hbm, out_hbm,
          srcall, dstall, bufa, bufb, paccv, attv, sema, semb, semw):
        cid = lax.axis_index("c")
        sid = lax.axis_index("s")
        wid = sid * NC + cid
        ebase = wid * EPW
        rbase = wid * (EPW * 16 // 128)
        pltpu.sync_copy(att_hbm, attv)
        pltpu.sync_copy(src_hbm.at[pl.ds(ebase, EPW)], srcall)
        pltpu.sync_copy(dst_hbm.at[pl.ds(ebase, EPW)], dstall)
        att6 = [attv[pl.ds(j * 16, 16)] * 0.6 for j in range(nj)]
        att4 = [attv[pl.ds(j * 16, 16)] * 0.4 for j in range(nj)]

        def ga(i, s):
            return pltpu.make_async_copy(
                a_hbm.at[srcall.at[pl.ds(i * ch1, ch1)]], bufa[s], sema[s])

        def gb(i, s):
            return pltpu.make_async_copy(
                b_hbm.at[dstall.at[pl.ds(i * ch1, ch1)]], bufb[s], semb[s])

        def wo(i, s):
            return pltpu.make_async_copy(
                paccv[s], out_hbm.at[pl.ds(rbase + i * pr1, pr1)], semw[s])

        def issue(i, s):
            ga(i, s).start()
            gb(i, s).start()

        def edge_loop(n, s):
            @pl.loop(0, n)
            def _edge(e):
                acc = jnp.zeros((16,), jnp.float32)
                for j in range(nj):
                    v = bufa[s][e, pl.ds(j * 16, 16)] + bufb[s][e, pl.ds(j * 16, 16)]
                    acc = acc + v * att6[j] + jnp.abs(v) * att4[j]
                paccv[s][e // 8, pl.ds((e % 8) * 16, 16)] = acc

        def compute(i, s):
            ga(i, s).wait()
            gb(i, s).wait()
            edge_loop(ch1, s)
            wo(i, s).start()

        issue(0, 0)
        issue(1, 1)
        compute(0, 0)
        issue(2, 0)
        compute(1, 1)

        @pl.loop(1, nf // 2)
        def _pair(p):
            i = 2 * p
            issue(i + 1, 1)
            wo(i - 2, 0).wait()
            compute(i, 0)
            issue(jnp.minimum(i + 2, nf - 1), 0)
            wo(i - 1, 1).wait()
            compute(i + 1, 1)

        ga(nf - 1, 0).wait()
        gb(nf - 1, 0).wait()
        pltpu.make_async_copy(
            a_hbm.at[srcall.at[pl.ds(nf * ch1, tl)]],
            bufa[0].at[pl.ds(0, tl)], sema[0]).start()
        pltpu.make_async_copy(
            b_hbm.at[dstall.at[pl.ds(nf * ch1, tl)]],
            bufb[0].at[pl.ds(0, tl)], semb[0]).start()
        wo(nf - 2, 0).wait()
        pltpu.make_async_copy(
            a_hbm.at[srcall.at[pl.ds(nf * ch1, tl)]],
            bufa[0].at[pl.ds(0, tl)], sema[0]).wait()
        pltpu.make_async_copy(
            b_hbm.at[dstall.at[pl.ds(nf * ch1, tl)]],
            bufb[0].at[pl.ds(0, tl)], semb[0]).wait()
        edge_loop(tl, 0)
        pltpu.make_async_copy(
            paccv[0].at[pl.ds(0, trow)],
            out_hbm.at[pl.ds(rbase + nf * pr1, trow)], semw[0]).start()
        wo(nf - 1, 1).wait()
        pltpu.make_async_copy(
            paccv[0].at[pl.ds(0, trow)],
            out_hbm.at[pl.ds(rbase + nf * pr1, trow)], semw[0]).wait()

    return k


def _make_pass2(co):
    cp = co + 16
    nj = cp // 16

    @functools.partial(
        pl.kernel, mesh=_MESH, compiler_params=_SC_PARAMS,
        out_type=jax.ShapeDtypeStruct((NC, N_NODES, cp), jnp.float32),
        scratch_types=[
            pltpu.VMEM((EPW,), jnp.int32),
            [pltpu.VMEM((CHUNK,), jnp.int32) for _ in range(2)],
            [pltpu.VMEM((PROWS, 128), jnp.float32) for _ in range(2)],
            [pltpu.VMEM((CHUNK, cp), jnp.float32) for _ in range(2)],
            pltpu.VMEM_SHARED((N_NODES, cp), jnp.float32),
            [pltpu.SemaphoreType.DMA for _ in range(2)],
            [pltpu.SemaphoreType.DMA for _ in range(2)],
            [pltpu.SemaphoreType.DMA for _ in range(2)],
        ],
    )
    def k(xlp_hbm, src_hbm, dst_hbm, exx_hbm, zeros_hbm, out_hbm,
          srcall, dstv, exv, rows, acc_sh, semg, seme, semd):
        cid = lax.axis_index("c")
        sid = lax.axis_index("s")
        wid = sid * NC + cid
        ebase = wid * EPW
        rbase = wid * (EPW * 16 // 128)
        r0 = sid * SUB_ROWS
        pltpu.sync_copy(src_hbm.at[pl.ds(ebase, EPW)], srcall)
        pltpu.sync_copy(zeros_hbm.at[pl.ds(r0, SUB_ROWS)],
                        acc_sh.at[pl.ds(r0, SUB_ROWS)])
        plsc.subcore_barrier()

        def gr(i, s):
            return pltpu.make_async_copy(
                xlp_hbm.at[srcall.at[pl.ds(i * CHUNK, CHUNK)]], rows[s], semg[s])

        def ge(i, s):
            return pltpu.make_async_copy(
                exx_hbm.at[pl.ds(rbase + i * PROWS, PROWS)], exv[s], seme[s])

        def gd(i, s):
            return pltpu.make_async_copy(
                dst_hbm.at[pl.ds(ebase + i * CHUNK, CHUNK)], dstv[s], semd[s])

        def issue(i, s):
            gr(i, s).start()
            ge(i, s).start()
            gd(i, s).start()

        def compute(i, s):
            gr(i, s).wait()
            ge(i, s).wait()
            gd(i, s).wait()

            @pl.loop(0, CHUNK)
            def _edge(e):
                exs = exv[s][e // 8, pl.ds((e % 8) * 16, 16)]
                for j in range(nj):
                    rows[s][e, pl.ds(j * 16, 16)] = rows[s][e, pl.ds(j * 16, 16)] * exs

            pltpu.sync_copy(rows[s], acc_sh.at[dstv[s]], add=True)

        issue(0, 0)
        issue(1, 1)
        compute(0, 0)
        issue(2, 0)
        compute(1, 1)

        @pl.loop(1, PAIRS)
        def _pair(p):
            i = 2 * p
            issue(i + 1, 1)
            compute(i, 0)
            issue(i + 2, 0)
            compute(i + 1, 1)

        compute(NCH - 1, 0)
        plsc.subcore_barrier()
        pltpu.sync_copy(acc_sh.at[pl.ds(r0, SUB_ROWS)],
                        out_hbm.at[cid, pl.ds(r0, SUB_ROWS)])

    return k


_MEAN_K = _make_mean_kernel()
_PASS1 = {128: _make_pass1(128), 64: _make_pass1(64)}
_PASS2 = {128: _make_pass2(128), 64: _make_pass2(64)}


def _edge_phase(a, b, xlp, src, dst, zeros_cp, att, co):
    pacc = _PASS1[co](a, b, src, dst, att)
    slog = xlp[:, co + 1]
    slogp = jnp.pad(slog, (0, 240), constant_values=-1e30).reshape(80, 128)
    exx, gmax = _mid(pacc, slogp)
    acc = _PASS2[co](xlp, src, dst, exx, zeros_cp)
    return acc, gmax


def kernel(x, edge_index, pos,
           Wl0, bl0, Wr0, br0, We0, att0, bo0,
           Wl1, bl1, Wr1, br1, We1, att1, bo1,
           Wl2, bl2, Wr2, br2, We2, att2, bo2):
    src = edge_index[0]
    dst = edge_index[1]
    posp = jnp.pad(pos, ((0, 0), (0, 13)))
    sums = _MEAN_K(posp, src, dst)
    mean_ea = (jnp.sum(sums, axis=0)[:DIM] / E_EDGES).reshape(1, DIM)
    z144 = jnp.zeros((N_NODES, 144), jnp.float32)
    z80 = jnp.zeros((N_NODES, 80), jnp.float32)
    hh = jnp.concatenate([x, pos], axis=1)
    a, b, xlp = _dense_pre(hh, pos, Wl0, bl0, Wr0, br0, We0, att0, mean_ea, 128)
    acc, gmax = _edge_phase(a, b, xlp, src, dst, z144, att0, 128)
    a, b, xlp = _fused(acc, xlp, gmax, bo0, pos,
                       Wl1, bl1, Wr1, br1, We1, att1, mean_ea, 128, 128)
    acc, gmax = _edge_phase(a, b, xlp, src, dst, z144, att1, 128)
    a, b, xlp = _fused(acc, xlp, gmax, bo1, pos,
                       Wl2, bl2, Wr2, br2, We2, att2, mean_ea, 128, 64)
    acc, gmax = _edge_phase(a, b, xlp, src, dst, z80, att2, 64)
    h = _final(acc, xlp, gmax, bo2, 64)
    return (h, edge_index, pos)
